# baseline (speedup 1.0000x reference)
import functools

import jax
import jax.numpy as jnp
from jax import lax
from jax.experimental import pallas as pl
from jax.experimental.pallas import tpu as pltpu
from jax.experimental.pallas import tpu_sc as plsc

N = 10000
D = 128
H = 128
C = 3
G = 64

NC = 2
NS = 16
NW = NC * NS
CHUNK = 128
NP = 10112
ROWS_PER = NP // NS

@functools.cache
def _mesh():
  return plsc.VectorSubcoreMesh(
      core_axis_name="c", subcore_axis_name="s", num_cores=NC, num_subcores=NS)


def _wid():
  return lax.axis_index("s") * NC + lax.axis_index("c")


def _sc_degree_body(dst_hbm, out_hbm, idx_v, hist):
  TE = idx_v.shape[0]
  w = _wid()
  pltpu.sync_copy(dst_hbm.at[w], idx_v)
  zeros = jnp.zeros((16,), jnp.float32)
  ones = jnp.ones((16,), jnp.float32)

  def zbody(i, _):
    hist[pl.ds(i * 16, 16)] = zeros
    return ()

  lax.fori_loop(0, NP // 16, zbody, (), unroll=False)

  def body(i, _):
    idx = idx_v[pl.ds(i * 16, 16)]
    plsc.addupdate_scatter(hist, [idx], ones)
    return ()

  lax.fori_loop(0, TE // 16, body, (), unroll=False)
  pltpu.sync_copy(hist, out_hbm.at[w])


def _sc_degree(dst_flat, T):
  kern = pl.kernel(
      _sc_degree_body,
      out_type=jax.ShapeDtypeStruct((NW, NP), jnp.float32),
      mesh=_mesh(),
      scratch_types=[
          pltpu.VMEM((T * CHUNK,), jnp.int32),
          pltpu.VMEM((NP,), jnp.float32),
      ],
      compiler_params=pltpu.CompilerParams(needs_layout_passes=False),
  )
  return kern(dst_flat)


SLOW_FRAC = 0.41


def _sc_scatter(y, src_idx, dst_idx, ts):
  tf = src_idx.shape[1]

  def body_fn(y_hbm, src_hbm, dst_hbm, zeros_hbm, out_hbm,
              src_v, dst_v, buf, acc, sem):
    cid = lax.axis_index("c")
    sid = lax.axis_index("s")
    pltpu.sync_copy(src_hbm.at[_wid()], src_v)
    pltpu.sync_copy(dst_hbm.at[_wid()], dst_v)
    pltpu.sync_copy(zeros_hbm, acc.at[pl.ds(sid * ROWS_PER, ROWS_PER)])
    plsc.subcore_barrier()

    def body(j, _):
      pltpu.async_copy(y_hbm.at[src_v.at[j]], buf, sem).wait()
      pltpu.sync_copy(buf, acc.at[dst_v.at[j]], add=True)
      return ()

    lax.fori_loop(0, jnp.where(cid == 0, tf, ts), body, (), unroll=False)
    plsc.subcore_barrier()
    sl = pl.ds(sid * ROWS_PER, ROWS_PER)
    pltpu.sync_copy(acc.at[sl], out_hbm.at[cid].at[sl])

  kern = pl.kernel(
      body_fn,
      out_type=jax.ShapeDtypeStruct((NC, NP, H), jnp.float32),
      mesh=_mesh(),
      scratch_types=[
          pltpu.VMEM((tf, CHUNK), jnp.int32),
          pltpu.VMEM((tf, CHUNK), jnp.int32),
          pltpu.VMEM((CHUNK, H), jnp.float32),
          pltpu.VMEM_SHARED((NP, H), jnp.float32),
          pltpu.SemaphoreType.DMA,
      ],
  )
  zeros = jnp.zeros((ROWS_PER, H), jnp.float32)
  return kern(y, src_idx, dst_idx, zeros)


def _tc_prep_body(x_ref, w_ref, degp_ref, y_ref, dinv_ref):
  x = x_ref[...]
  mu = jnp.mean(x, axis=0, keepdims=True)
  xc = x - mu
  var = jnp.sum(xc * xc, axis=0, keepdims=True) / (N - 1)
  xn = xc / (jnp.sqrt(var) + 1e-6)
  xw = jnp.dot(xn, w_ref[...], preferred_element_type=jnp.float32)
  deg = lax.dot_general(
      degp_ref[...], jnp.ones((NW, 1), jnp.float32),
      (((0,), (0,)), ((), ())), preferred_element_type=jnp.float32) + 1.0
  rows = lax.broadcasted_iota(jnp.int32, (NP, 1), 0)
  dinv = jnp.where(rows < N, lax.rsqrt(deg), 0.0)
  dinv_ref[...] = dinv
  y_ref[pl.ds(0, N), :] = dinv[:N] * xw
  y_ref[pl.ds(N, NP - N), :] = jnp.zeros((NP - N, H), jnp.float32)


def _tc_prep(x, W1, degp):
  return pl.pallas_call(
      _tc_prep_body,
      out_shape=[
          jax.ShapeDtypeStruct((NP, H), jnp.float32),
          jax.ShapeDtypeStruct((NP, 1), jnp.float32),
      ],
  )(x, W1, degp)


def _tc_mid_body(s_ref, y_ref, dinv_ref, b_ref, w_ref, out_ref):
  s = s_ref[0] + s_ref[1] + y_ref[...]
  h = jnp.maximum(dinv_ref[...] * s + b_ref[...], 0.0)
  out_ref[...] = dinv_ref[...] * jnp.dot(
      h, w_ref[...], preferred_element_type=jnp.float32)


def _tc_mid(s, y, dinv, b, W):
  return pl.pallas_call(
      _tc_mid_body,
      out_shape=jax.ShapeDtypeStruct((NP, H), jnp.float32),
  )(s, y, dinv, b.reshape(1, H), W)


def _tc_final_body(s_ref, y_ref, dinv_ref, b_ref, batch_ref, wc_ref, bc_ref,
                   out_ref):
  s = s_ref[0] + s_ref[1] + y_ref[...]
  h = jnp.maximum(dinv_ref[...] * s + b_ref[...], 0.0)
  gids = lax.broadcasted_iota(jnp.int32, (G, NP), 0)
  m = (batch_ref[...] == gids).astype(jnp.float32)
  cnt = jnp.sum(m, axis=1, keepdims=True)
  pooled = jnp.dot(m, h, preferred_element_type=jnp.float32)
  pooled = pooled / jnp.maximum(cnt, 1.0)
  out_ref[...] = jnp.dot(
      pooled, wc_ref[...], preferred_element_type=jnp.float32) + bc_ref[...]


def _tc_final(s, y, dinv, b, batch, Wc, bc):
  return pl.pallas_call(
      _tc_final_body,
      out_shape=jax.ShapeDtypeStruct((G, C), jnp.float32),
  )(s, y, dinv, b.reshape(1, H), batch, Wc, bc.reshape(1, C))


def kernel(x, edge_index, batch, W1, b1, W2, b2, Wc, bc):
  E = edge_index.shape[1]
  T = -(-E // (NW * CHUNK))
  EP = NW * T * CHUNK
  pad = jnp.full((EP - E,), N, jnp.int32)
  dst_flat = jnp.concatenate([edge_index[1], pad]).reshape(NW, T * CHUNK)
  batch_p = jnp.concatenate(
      [batch, jnp.full((NP - N,), G, jnp.int32)]).reshape(1, NP)

  ntc = -(-E // CHUNK)
  per_pair = -(-ntc // NS)
  ts = max(1, int(per_pair * SLOW_FRAC))
  tf = per_pair - ts
  ncap = NS * (tf + ts)
  cpad = jnp.full((ncap * CHUNK - E,), N, jnp.int32)

  def layout(idx):
    c = jnp.concatenate([idx, cpad]).reshape(ncap, CHUNK)
    fast = c[:NS * tf].reshape(NS, 1, tf, CHUNK)
    slow = c[NS * tf:].reshape(NS, 1, ts, CHUNK)
    slow = jnp.concatenate(
        [slow, jnp.full((NS, 1, tf - ts, CHUNK), N, jnp.int32)], axis=2)
    return jnp.concatenate([fast, slow], axis=1).reshape(NW, tf, CHUNK)

  src_b = layout(edge_index[0])
  dst_b = layout(edge_index[1])

  degp = _sc_degree(dst_flat, T)
  y1, dinv = _tc_prep(x, W1, degp)
  s1 = _sc_scatter(y1, src_b, dst_b, ts)
  y2 = _tc_mid(s1, y1, dinv, b1, W2)
  s2 = _sc_scatter(y2, src_b, dst_b, ts)
  return _tc_final(s2, y2, dinv, b2, batch_p, Wc, bc)


# speedup vs baseline: 1.0169x; 1.0169x over previous
import functools

import jax
import jax.numpy as jnp
from jax import lax
from jax.experimental import pallas as pl
from jax.experimental.pallas import tpu as pltpu
from jax.experimental.pallas import tpu_sc as plsc

N = 10000
D = 128
H = 128
C = 3
G = 64

NC = 2
NS = 16
NW = NC * NS
CHUNK = 128
NP = 10112
ROWS_PER = NP // NS

@functools.cache
def _mesh():
  return plsc.VectorSubcoreMesh(
      core_axis_name="c", subcore_axis_name="s", num_cores=NC, num_subcores=NS)


def _wid():
  return lax.axis_index("s") * NC + lax.axis_index("c")


def _sc_degree_body(dst_hbm, out_hbm, idx_v, hist):
  TE = idx_v.shape[0]
  w = _wid()
  pltpu.sync_copy(dst_hbm.at[w], idx_v)
  zeros = jnp.zeros((16,), jnp.float32)
  ones = jnp.ones((16,), jnp.float32)

  def zbody(i, _):
    hist[pl.ds(i * 16, 16)] = zeros
    return ()

  lax.fori_loop(0, NP // 16, zbody, (), unroll=False)

  def body(i, _):
    idx = idx_v[pl.ds(i * 16, 16)]
    plsc.addupdate_scatter(hist, [idx], ones)
    return ()

  lax.fori_loop(0, TE // 16, body, (), unroll=False)
  pltpu.sync_copy(hist, out_hbm.at[w])


def _sc_degree(dst_flat, T):
  kern = pl.kernel(
      _sc_degree_body,
      out_type=jax.ShapeDtypeStruct((NW, NP), jnp.float32),
      mesh=_mesh(),
      scratch_types=[
          pltpu.VMEM((T * CHUNK,), jnp.int32),
          pltpu.VMEM((NP,), jnp.float32),
      ],
      compiler_params=pltpu.CompilerParams(needs_layout_passes=False),
  )
  return kern(dst_flat)


SLOW_FRAC = 0.395


def _sc_scatter(y, src_idx, dst_idx, ts):
  tf = src_idx.shape[1]

  def body_fn(y_hbm, src_hbm, dst_hbm, zeros_hbm, out_hbm,
              src_v, dst_v, buf, acc, sem):
    cid = lax.axis_index("c")
    sid = lax.axis_index("s")
    pltpu.sync_copy(src_hbm.at[_wid()], src_v)
    pltpu.sync_copy(dst_hbm.at[_wid()], dst_v)
    pltpu.sync_copy(zeros_hbm, acc.at[pl.ds(sid * ROWS_PER, ROWS_PER)])
    plsc.subcore_barrier()

    def body(j, _):
      pltpu.async_copy(y_hbm.at[src_v.at[j]], buf, sem).wait()
      pltpu.sync_copy(buf, acc.at[dst_v.at[j]], add=True)
      return ()

    lax.fori_loop(0, jnp.where(cid == 0, tf, ts), body, (), unroll=False)
    plsc.subcore_barrier()
    sl = pl.ds(sid * ROWS_PER, ROWS_PER)
    pltpu.sync_copy(acc.at[sl], out_hbm.at[cid].at[sl])

  kern = pl.kernel(
      body_fn,
      out_type=jax.ShapeDtypeStruct((NC, NP, H), jnp.float32),
      mesh=_mesh(),
      scratch_types=[
          pltpu.VMEM((tf, CHUNK), jnp.int32),
          pltpu.VMEM((tf, CHUNK), jnp.int32),
          pltpu.VMEM((CHUNK, H), jnp.float32),
          pltpu.VMEM_SHARED((NP, H), jnp.float32),
          pltpu.SemaphoreType.DMA,
      ],
  )
  zeros = jnp.zeros((ROWS_PER, H), jnp.float32)
  return kern(y, src_idx, dst_idx, zeros)


def _tc_prep_body(x_ref, w_ref, degp_ref, y_ref, dinv_ref):
  x = x_ref[...]
  mu = jnp.mean(x, axis=0, keepdims=True)
  xc = x - mu
  var = jnp.sum(xc * xc, axis=0, keepdims=True) / (N - 1)
  xn = xc / (jnp.sqrt(var) + 1e-6)
  xw = jnp.dot(xn, w_ref[...], preferred_element_type=jnp.float32)
  deg = lax.dot_general(
      degp_ref[...], jnp.ones((NW, 1), jnp.float32),
      (((0,), (0,)), ((), ())), preferred_element_type=jnp.float32) + 1.0
  rows = lax.broadcasted_iota(jnp.int32, (NP, 1), 0)
  dinv = jnp.where(rows < N, lax.rsqrt(deg), 0.0)
  dinv_ref[...] = dinv
  y_ref[pl.ds(0, N), :] = dinv[:N] * xw
  y_ref[pl.ds(N, NP - N), :] = jnp.zeros((NP - N, H), jnp.float32)


def _tc_prep(x, W1, degp):
  return pl.pallas_call(
      _tc_prep_body,
      out_shape=[
          jax.ShapeDtypeStruct((NP, H), jnp.float32),
          jax.ShapeDtypeStruct((NP, 1), jnp.float32),
      ],
  )(x, W1, degp)


def _tc_mid_body(s_ref, y_ref, dinv_ref, b_ref, w_ref, out_ref):
  s = s_ref[0] + s_ref[1] + y_ref[...]
  h = jnp.maximum(dinv_ref[...] * s + b_ref[...], 0.0)
  out_ref[...] = dinv_ref[...] * jnp.dot(
      h, w_ref[...], preferred_element_type=jnp.float32)


def _tc_mid(s, y, dinv, b, W):
  return pl.pallas_call(
      _tc_mid_body,
      out_shape=jax.ShapeDtypeStruct((NP, H), jnp.float32),
  )(s, y, dinv, b.reshape(1, H), W)


def _tc_final_body(s_ref, y_ref, dinv_ref, b_ref, batch_ref, wc_ref, bc_ref,
                   out_ref):
  s = s_ref[0] + s_ref[1] + y_ref[...]
  h = jnp.maximum(dinv_ref[...] * s + b_ref[...], 0.0)
  gids = lax.broadcasted_iota(jnp.int32, (G, NP), 0)
  m = (batch_ref[...] == gids).astype(jnp.float32)
  cnt = jnp.sum(m, axis=1, keepdims=True)
  pooled = jnp.dot(m, h, preferred_element_type=jnp.float32)
  pooled = pooled / jnp.maximum(cnt, 1.0)
  out_ref[...] = jnp.dot(
      pooled, wc_ref[...], preferred_element_type=jnp.float32) + bc_ref[...]


def _tc_final(s, y, dinv, b, batch, Wc, bc):
  return pl.pallas_call(
      _tc_final_body,
      out_shape=jax.ShapeDtypeStruct((G, C), jnp.float32),
  )(s, y, dinv, b.reshape(1, H), batch, Wc, bc.reshape(1, C))


def kernel(x, edge_index, batch, W1, b1, W2, b2, Wc, bc):
  E = edge_index.shape[1]
  T = -(-E // (NW * CHUNK))
  EP = NW * T * CHUNK
  pad = jnp.full((EP - E,), N, jnp.int32)
  dst_flat = jnp.concatenate([edge_index[1], pad]).reshape(NW, T * CHUNK)
  batch_p = jnp.concatenate(
      [batch, jnp.full((NP - N,), G, jnp.int32)]).reshape(1, NP)

  ntc = -(-E // CHUNK)
  per_pair = -(-ntc // NS)
  ts = max(1, int(per_pair * SLOW_FRAC))
  tf = per_pair - ts
  ncap = NS * (tf + ts)
  cpad = jnp.full((ncap * CHUNK - E,), N, jnp.int32)

  def layout(idx):
    c = jnp.concatenate([idx, cpad]).reshape(ncap, CHUNK)
    fast = c[:NS * tf].reshape(NS, 1, tf, CHUNK)
    slow = c[NS * tf:].reshape(NS, 1, ts, CHUNK)
    slow = jnp.concatenate(
        [slow, jnp.full((NS, 1, tf - ts, CHUNK), N, jnp.int32)], axis=2)
    return jnp.concatenate([fast, slow], axis=1).reshape(NW, tf, CHUNK)

  src_b = layout(edge_index[0])
  dst_b = layout(edge_index[1])

  degp = _sc_degree(dst_flat, T)
  y1, dinv = _tc_prep(x, W1, degp)
  s1 = _sc_scatter(y1, src_b, dst_b, ts)
  y2 = _tc_mid(s1, y1, dinv, b1, W2)
  s2 = _sc_scatter(y2, src_b, dst_b, ts)
  return _tc_final(s2, y2, dinv, b2, batch_p, Wc, bc)
